# Initial kernel scaffold; baseline (speedup 1.0000x reference)
import functools

import jax
import jax.numpy as jnp
from jax import lax
from jax.experimental import pallas as pl
from jax.experimental.pallas import tpu as pltpu
from jax.experimental.pallas import tpu_sc as plsc

VOCAB = 1000000
EMBED_DIM = 32
BATCH = 4096
HIST = 200

B = BATCH * HIST
NC = 2
NS = 16
NW = NC * NS
BPW = B // NW
CHUNK = 1024
NCHUNK = BPW // CHUNK

_mesh = plsc.VectorSubcoreMesh(core_axis_name="c", subcore_axis_name="s")


@functools.partial(
    pl.kernel,
    mesh=_mesh,
    out_type=jax.ShapeDtypeStruct((B, EMBED_DIM), jnp.float32),
    scratch_types=[
        pltpu.VMEM((CHUNK,), jnp.int32),
        pltpu.VMEM((CHUNK, EMBED_DIM), jnp.float32),
        pltpu.SemaphoreType.DMA,
    ],
)
def _sc_gather(x_hbm, table_hbm, out_hbm, idx_v, rows_v, sem):
    wid = lax.axis_index("s") * NC + lax.axis_index("c")
    base = wid * BPW

    def body(i, carry):
        off = base + i * CHUNK
        pltpu.sync_copy(x_hbm.at[pl.ds(off, CHUNK)], idx_v)
        pltpu.async_copy(table_hbm.at[idx_v], rows_v, sem).wait()
        pltpu.sync_copy(rows_v, out_hbm.at[pl.ds(off, CHUNK)])
        return carry

    lax.fori_loop(0, NCHUNK, body, 0)


def kernel(x, weight):
    flat = x.reshape(-1)
    out = _sc_gather(flat, weight)
    return out.reshape(x.shape + (weight.shape[1],))


# speedup vs baseline: 1.4596x; 1.4596x over previous
import functools

import jax
import jax.numpy as jnp
from jax import lax
from jax.experimental import pallas as pl
from jax.experimental.pallas import tpu as pltpu
from jax.experimental.pallas import tpu_sc as plsc

VOCAB = 1000000
EMBED_DIM = 32
BATCH = 4096
HIST = 200

B = BATCH * HIST
NC = 2
NS = 16
NW = NC * NS
BPW = B // NW
CHUNK = 1024
NCHUNK = BPW // CHUNK

_mesh = plsc.VectorSubcoreMesh(core_axis_name="c", subcore_axis_name="s")


@functools.partial(
    pl.kernel,
    mesh=_mesh,
    compiler_params=pltpu.CompilerParams(use_tc_tiling_on_sc=False),
    out_type=jax.ShapeDtypeStruct((B, EMBED_DIM), jnp.float32),
    scratch_types=[
        pltpu.VMEM((CHUNK,), jnp.int32),
        pltpu.VMEM((CHUNK, EMBED_DIM), jnp.float32),
        pltpu.SemaphoreType.DMA,
    ],
)
def _sc_gather(x_hbm, table_hbm, out_hbm, idx_v, rows_v, sem):
    wid = lax.axis_index("s") * NC + lax.axis_index("c")
    base = wid * BPW

    def body(i, carry):
        off = base + i * CHUNK
        pltpu.sync_copy(x_hbm.at[pl.ds(off, CHUNK)], idx_v)
        pltpu.async_copy(table_hbm.at[idx_v], rows_v, sem).wait()
        pltpu.sync_copy(rows_v, out_hbm.at[pl.ds(off, CHUNK)])
        return carry

    lax.fori_loop(0, NCHUNK, body, 0)


def kernel(x, weight):
    flat = x.reshape(-1)
    out = _sc_gather(flat, weight)
    return out.reshape(x.shape + (weight.shape[1],))


# speedup vs baseline: 1.4995x; 1.0273x over previous
import functools

import jax
import jax.numpy as jnp
from jax import lax
from jax.experimental import pallas as pl
from jax.experimental.pallas import tpu as pltpu
from jax.experimental.pallas import tpu_sc as plsc

VOCAB = 1000000
EMBED_DIM = 32
BATCH = 4096
HIST = 200

B = BATCH * HIST
NC = 2
NS = 16
NW = NC * NS
BPW = B // NW
CHUNK = 800
NCHUNK = BPW // CHUNK
NBUF = 2

_mesh = plsc.VectorSubcoreMesh(core_axis_name="c", subcore_axis_name="s")


@functools.partial(
    pl.kernel,
    mesh=_mesh,
    compiler_params=pltpu.CompilerParams(use_tc_tiling_on_sc=False),
    out_type=jax.ShapeDtypeStruct((B, EMBED_DIM), jnp.float32),
    scratch_types=(
        [pltpu.VMEM((CHUNK,), jnp.int32)] * NBUF
        + [pltpu.VMEM((CHUNK, EMBED_DIM), jnp.float32)] * NBUF
        + [pltpu.SemaphoreType.DMA] * (2 * NBUF)
    ),
)
def _sc_gather(x_hbm, table_hbm, out_hbm, *refs):
    idx = refs[:NBUF]
    rows = refs[NBUF:2 * NBUF]
    sem_g = refs[2 * NBUF:3 * NBUF]
    sem_o = refs[3 * NBUF:4 * NBUF]

    wid = lax.axis_index("s") * NC + lax.axis_index("c")
    base = wid * BPW

    def stage_idx(g, b):
        pltpu.sync_copy(x_hbm.at[pl.ds(base + g * CHUNK, CHUNK)], idx[b])

    def start_gather(b):
        pltpu.async_copy(table_hbm.at[idx[b]], rows[b], sem_g[b])

    def wait_gather(b):
        pltpu.make_async_copy(table_hbm.at[pl.ds(0, CHUNK)], rows[b], sem_g[b]).wait()

    def start_out(g, b):
        pltpu.async_copy(rows[b], out_hbm.at[pl.ds(base + g * CHUNK, CHUNK)], sem_o[b])

    def wait_out(g, b):
        pltpu.make_async_copy(
            rows[b], out_hbm.at[pl.ds(base + g * CHUNK, CHUNK)], sem_o[b]
        ).wait()

    for b in range(NBUF):
        stage_idx(b, b)
        start_gather(b)

    def body(p, carry):
        for b in range(NBUF):
            g = NBUF * p + b
            wait_gather(b)
            start_out(g, b)
            stage_idx(g + NBUF, b)
            wait_out(g, b)
            start_gather(b)
        return carry

    lax.fori_loop(0, (NCHUNK - NBUF) // NBUF, body, 0)

    for b in range(NBUF):
        g = NCHUNK - NBUF + b
        wait_gather(b)
        start_out(g, b)
    for b in range(NBUF):
        g = NCHUNK - NBUF + b
        wait_out(g, b)


def kernel(x, weight):
    flat = x.reshape(-1)
    out = _sc_gather(flat, weight)
    return out.reshape(x.shape + (weight.shape[1],))


# speedup vs baseline: 1.5021x; 1.0017x over previous
import functools

import jax
import jax.numpy as jnp
from jax import lax
from jax.experimental import pallas as pl
from jax.experimental.pallas import tpu as pltpu
from jax.experimental.pallas import tpu_sc as plsc

VOCAB = 1000000
EMBED_DIM = 32
BATCH = 4096
HIST = 200

B = BATCH * HIST
NC = 2
NS = 16
NW = NC * NS
BPW = B // NW
CHUNK = 640
NCHUNK = BPW // CHUNK
NBUF = 4

_mesh = plsc.VectorSubcoreMesh(core_axis_name="c", subcore_axis_name="s")


@functools.partial(
    pl.kernel,
    mesh=_mesh,
    compiler_params=pltpu.CompilerParams(use_tc_tiling_on_sc=False),
    out_type=jax.ShapeDtypeStruct((B, EMBED_DIM), jnp.float32),
    scratch_types=(
        [pltpu.VMEM((CHUNK,), jnp.int32)] * NBUF
        + [pltpu.VMEM((CHUNK, EMBED_DIM), jnp.float32)] * NBUF
        + [pltpu.SemaphoreType.DMA] * (2 * NBUF)
    ),
)
def _sc_gather(x_hbm, table_hbm, out_hbm, *refs):
    idx = refs[:NBUF]
    rows = refs[NBUF:2 * NBUF]
    sem_g = refs[2 * NBUF:3 * NBUF]
    sem_o = refs[3 * NBUF:4 * NBUF]

    wid = lax.axis_index("s") * NC + lax.axis_index("c")
    base = wid * BPW

    def stage_idx(g, b):
        pltpu.sync_copy(x_hbm.at[pl.ds(base + g * CHUNK, CHUNK)], idx[b])

    def start_gather(b):
        pltpu.async_copy(table_hbm.at[idx[b]], rows[b], sem_g[b])

    def wait_gather(b):
        pltpu.make_async_copy(table_hbm.at[pl.ds(0, CHUNK)], rows[b], sem_g[b]).wait()

    def start_out(g, b):
        pltpu.async_copy(rows[b], out_hbm.at[pl.ds(base + g * CHUNK, CHUNK)], sem_o[b])

    def wait_out(g, b):
        pltpu.make_async_copy(
            rows[b], out_hbm.at[pl.ds(base + g * CHUNK, CHUNK)], sem_o[b]
        ).wait()

    for b in range(NBUF):
        stage_idx(b, b)
        start_gather(b)

    def body(p, carry):
        for b in range(NBUF):
            g = NBUF * p + b
            wait_gather(b)
            start_out(g, b)
            stage_idx(g + NBUF, b)
            wait_out(g, b)
            start_gather(b)
        return carry

    lax.fori_loop(0, (NCHUNK - NBUF) // NBUF, body, 0)

    for b in range(NBUF):
        g = NCHUNK - NBUF + b
        wait_gather(b)
        start_out(g, b)
    for b in range(NBUF):
        g = NCHUNK - NBUF + b
        wait_out(g, b)


def kernel(x, weight):
    flat = x.reshape(-1)
    out = _sc_gather(flat, weight)
    return out.reshape(x.shape + (weight.shape[1],))


# speedup vs baseline: 1.5525x; 1.0336x over previous
import functools

import jax
import jax.numpy as jnp
from jax import lax
from jax.experimental import pallas as pl
from jax.experimental.pallas import tpu as pltpu
from jax.experimental.pallas import tpu_sc as plsc

VOCAB = 1000000
EMBED_DIM = 32
BATCH = 4096
HIST = 200

B = BATCH * HIST
NC = 2
NS = 16
NW = NC * NS
BL = BATCH // NW
CHUNK = HIST
NBUF = 4
QG = HIST // 4

_mesh = plsc.VectorSubcoreMesh(core_axis_name="c", subcore_axis_name="s")


@functools.partial(
    pl.kernel,
    mesh=_mesh,
    compiler_params=pltpu.CompilerParams(use_tc_tiling_on_sc=False),
    out_type=jax.ShapeDtypeStruct((NW, HIST, BL, EMBED_DIM), jnp.float32),
    scratch_types=(
        [pltpu.VMEM((CHUNK,), jnp.int32)] * NBUF
        + [pltpu.VMEM((CHUNK, EMBED_DIM), jnp.float32)] * NBUF
        + [pltpu.SemaphoreType.DMA] * (2 * NBUF)
    ),
)
def _sc_gather(x_hbm, table_hbm, out_hbm, *refs):
    idx = refs[:NBUF]
    rows = refs[NBUF:2 * NBUF]
    sem_g = refs[2 * NBUF:3 * NBUF]
    sem_o = refs[3 * NBUF:4 * NBUF]

    wid = lax.axis_index("s") * NC + lax.axis_index("c")
    base = wid * BL * HIST

    def stage_idx(l, b):
        pltpu.sync_copy(x_hbm.at[pl.ds(base + l * CHUNK, CHUNK)], idx[b])

    def start_gather(b):
        pltpu.async_copy(table_hbm.at[idx[b]], rows[b], sem_g[b])

    def wait_gather(b):
        pltpu.make_async_copy(table_hbm.at[idx[b]], rows[b], sem_g[b]).wait()

    def start_out(l, b):
        pltpu.async_copy(rows[b], out_hbm.at[wid, :, l, :], sem_o[b])

    def wait_out(l, b):
        pltpu.make_async_copy(rows[b], out_hbm.at[wid, :, l, :], sem_o[b]).wait()

    for b in range(NBUF):
        stage_idx(b, b)
        start_gather(b)

    def body(p, carry):
        for b in range(NBUF):
            l = NBUF * p + b
            wait_gather(b)
            start_out(l, b)
            stage_idx(l + NBUF, b)
            wait_out(l, b)
            start_gather(b)
        return carry

    lax.fori_loop(0, (BL - NBUF) // NBUF, body, 0)

    for b in range(NBUF):
        l = BL - NBUF + b
        wait_gather(b)
        start_out(l, b)
    for b in range(NBUF):
        l = BL - NBUF + b
        wait_out(l, b)


HB = 25


def _tc_body(in_ref, out_ref):
    for i in range(HB):
        m = in_ref[i].reshape(BL, EMBED_DIM)
        out_ref[pl.ds(4 * i, 4), 0] = m.T.reshape(4, 8, 128)


def _tc_transpose(rows3):
    return pl.pallas_call(
        _tc_body,
        grid=(NW, HIST // HB),
        in_specs=[pl.BlockSpec((HB, 32, 128),
                               lambda bt, hb: (bt * (HIST // HB) + hb, 0, 0))],
        out_specs=pl.BlockSpec((4 * HB, 1, 8, 128),
                               lambda bt, hb: (hb, bt, 0, 0)),
        out_shape=jax.ShapeDtypeStruct((4 * HIST, NW, 8, 128), jnp.float32),
    )(rows3)


def kernel(x, weight):
    flat = x.reshape(-1)
    slabs = _sc_gather(flat, weight)
    rows3 = slabs.reshape(NW * HIST, 32, 128)
    out4 = _tc_transpose(rows3)
    out5 = out4.reshape(HIST, 4, NW, 8, 128)
    return out5.transpose(2, 4, 0, 1, 3).reshape(BATCH, HIST, EMBED_DIM)
